# baseline (speedup 1.0000x reference)
import functools

import jax
import jax.numpy as jnp
from jax import lax
from jax.experimental import pallas as pl
from jax.experimental.pallas import tpu as pltpu
import jax.experimental.pallas.tpu_sc as plsc

N = 10000
E = 160000
G = 128
L = 16
NCORES = 2
NSUB = 16
CCH = 40
NT = 624
TAIL = N - NT * NSUB
ZR = 16


def _make_conv(S):
    EC = E // NCORES
    ET = EC // NSUB
    NCHUNK = ET // CCH
    mesh = plsc.VectorSubcoreMesh(core_axis_name="c", subcore_axis_name="s")

    @functools.partial(
        pl.kernel,
        out_type=jax.ShapeDtypeStruct((NCORES * S * N, 128), jnp.float32),
        mesh=mesh,
        scratch_types=[
            pltpu.VMEM((CCH,), jnp.int32),
            pltpu.VMEM((CCH,), jnp.int32),
            pltpu.VMEM((CCH, 128), jnp.float32),
            pltpu.VMEM((CCH, 128), jnp.float32),
            pltpu.VMEM((CCH, 128), jnp.float32),
            pltpu.VMEM((ZR, 128), jnp.float32),
            pltpu.VMEM_SHARED((N, 128), jnp.float32),
            pltpu.SemaphoreType.DMA,
        ],
    )
    def conv(h_ref, e_ref, srcs_ref, dst_ref, out_ref,
             si_v, di_v, x_v, e_v, m_v, z_v, acc, sem):
        c = lax.axis_index("c")
        t = lax.axis_index("s")
        zero16 = jnp.zeros((L,), jnp.float32)
        for r in range(ZR):
            for f in range(128 // L):
                z_v[r, pl.ds(f * L, L)] = zero16

        def slab_body(s, _):
            def zb(i, _):
                pltpu.sync_copy(z_v, acc.at[pl.ds(t * NT + i * ZR, ZR)])
                return 0
            lax.fori_loop(0, NT // ZR, zb, 0)

            @pl.when(t == NSUB - 1)
            def _():
                pltpu.sync_copy(z_v, acc.at[pl.ds(NT * NSUB, TAIL)])
            plsc.subcore_barrier()

            ebase0 = c * EC + t * ET

            def chunk(i, _):
                eb = ebase0 + i * CCH
                pltpu.sync_copy(srcs_ref.at[pl.ds(s * E + eb, CCH)], si_v)
                pltpu.sync_copy(dst_ref.at[pl.ds(eb, CCH)], di_v)
                pltpu.async_copy(h_ref.at[si_v], x_v, sem).wait()
                pltpu.sync_copy(e_ref.at[pl.ds(s * E + eb, CCH)], e_v)
                for cc in range(CCH):
                    for f in range(128 // L):
                        sl = pl.ds(f * L, L)
                        m_v[cc, sl] = jnp.maximum(x_v[cc, sl] + e_v[cc, sl],
                                                  0.0)
                pltpu.sync_copy(m_v, acc.at[di_v], add=True)
                return 0
            lax.fori_loop(0, NCHUNK, chunk, 0)
            plsc.subcore_barrier()

            ob0 = (c * S + s) * N
            pltpu.sync_copy(acc.at[pl.ds(t * NT, NT)],
                            out_ref.at[pl.ds(ob0 + t * NT, NT)])

            @pl.when(t == NSUB - 1)
            def _():
                pltpu.sync_copy(acc.at[pl.ds(NT * NSUB, TAIL)],
                                out_ref.at[pl.ds(ob0 + NT * NSUB, TAIL)])
            plsc.subcore_barrier()
            return 0
        lax.fori_loop(0, S, slab_body, 0)

    return conv


def _proj(ea_p, wp, bp, S):
    BE = 2000

    def body(ea_ref, w_ref, b_ref, o_ref):
        o_ref[0] = (jnp.dot(ea_ref[...], w_ref[0],
                            preferred_element_type=jnp.float32)
                    + b_ref[0])

    return pl.pallas_call(
        body,
        grid=(S, E // BE),
        in_specs=[
            pl.BlockSpec((BE, 16), lambda s, e: (e, 0)),
            pl.BlockSpec((1, 16, 128), lambda s, e: (s, 0, 0)),
            pl.BlockSpec((1, 1, 128), lambda s, e: (s, 0, 0)),
        ],
        out_specs=pl.BlockSpec((1, BE, 128), lambda s, e: (s, e, 0)),
        out_shape=jax.ShapeDtypeStruct((S, E, 128), jnp.float32),
    )(ea_p, wp, bp)


def _dense(h, agg, w, b, eps_arr, s_in, s_out):
    BN = 1000

    def body(eps_ref, h_ref, a_ref, w_ref, b_ref, o_ref):
        k = 1.0 + eps_ref[0, 0]
        acc = jnp.zeros((BN, s_out * 128), jnp.float32)
        for s in range(s_in):
            xs = h_ref[s] * k + a_ref[0, s] + a_ref[1, s]
            acc = acc + jnp.dot(xs, w_ref[s],
                                preferred_element_type=jnp.float32)
        z = acc + b_ref[0]
        z = jnp.where(z > 0, z, jnp.exp(jnp.minimum(z, 0.0)) - 1.0)
        for s2 in range(s_out):
            o_ref[s2] = z[:, s2 * 128:(s2 + 1) * 128]

    return pl.pallas_call(
        body,
        grid=(N // BN,),
        in_specs=[
            pl.BlockSpec(memory_space=pltpu.SMEM),
            pl.BlockSpec((s_in, BN, 128), lambda nb: (0, nb, 0)),
            pl.BlockSpec((2, s_in, BN, 128), lambda nb: (0, 0, nb, 0)),
            pl.BlockSpec((s_in, 128, s_out * 128), lambda nb: (0, 0, 0)),
            pl.BlockSpec((1, s_out * 128), lambda nb: (0, 0)),
        ],
        out_specs=pl.BlockSpec((s_out, BN, 128), lambda nb: (0, nb, 0)),
        out_shape=jax.ShapeDtypeStruct((s_out, N, 128), jnp.float32),
    )(eps_arr, h, agg, w, b)


def _make_segmax(S):
    gpw = G // (NCORES * NSUB)
    mesh = plsc.VectorSubcoreMesh(core_axis_name="c", subcore_axis_name="s")

    @functools.partial(
        pl.kernel,
        out_type=jax.ShapeDtypeStruct((G * S * 8, 128), jnp.float32),
        mesh=mesh,
        scratch_types=[
            pltpu.VMEM((272,), jnp.int32),
            pltpu.VMEM((8, 128), jnp.float32),
            pltpu.VMEM((8, 128), jnp.float32),
            pltpu.SemaphoreType.DMA,
        ],
    )
    def segmax(h_ref, b_ref, out_ref, bvm, row_v, acc_v, sem):
        c = lax.axis_index("c")
        t = lax.axis_index("s")
        wid = c * NSUB + t
        pltpu.sync_copy(b_ref, bvm)
        base = wid * gpw
        bvec = bvm[pl.ds(wid * 8, L)]

        def extract(j):
            return bvec[j]

        neg = jnp.full((L,), -jnp.inf, jnp.float32)
        for g in range(gpw):
            s0 = extract(g)
            s1 = extract(g + 1)
            s0a = (s0 // 8) * 8
            nch = (s1 - s0a + 7) // 8

            def slab_body(s, _):
                def chunk(i, accs):
                    r0 = jnp.minimum(s0a + i * 8, N - 8)
                    pltpu.async_copy(h_ref.at[pl.ds(s * N + r0, 8)],
                                     row_v, sem).wait()
                    out = list(accs)
                    for k in range(8):
                        rid = r0 + k
                        valid = jnp.logical_and(rid >= s0, rid < s1)
                        for f in range(128 // L):
                            v = jnp.where(valid, row_v[k, pl.ds(f * L, L)],
                                          neg)
                            out[f] = jnp.maximum(out[f], v)
                    return tuple(out)
                accs = lax.fori_loop(0, nch, chunk, (neg,) * (128 // L))
                for k in range(8):
                    for f in range(128 // L):
                        acc_v[k, pl.ds(f * L, L)] = accs[f]
                pltpu.sync_copy(
                    acc_v, out_ref.at[pl.ds(((base + g) * S + s) * 8, 8)])
                return 0
            lax.fori_loop(0, S, slab_body, 0)

    return segmax


def _pad2(a, rows, cols):
    return jnp.pad(a, ((0, rows - a.shape[0]), (0, cols - a.shape[1])))


def kernel(x1, edge_index, edge_attr, batch,
           fc1_W, fc1_b, fc2_W, fc2_b, fc3_W, fc3_b,
           gin1_W, gin1_b, eps1, gin2_W, gin2_b, eps2,
           gin3_W, gin3_b, eps3):
    src = edge_index[0].astype(jnp.int32)
    dst = edge_index[1].astype(jnp.int32)

    ea_p = _pad2(edge_attr, E, 16)

    def prep_fc(w, b, S):
        wp = _pad2(w, 16, S * 128).reshape(16, S, 128).transpose(1, 0, 2)
        bp = jnp.pad(b, (0, S * 128 - b.shape[0])).reshape(S, 1, 128)
        return wp, bp

    w1p, b1p = prep_fc(fc1_W, fc1_b, 1)
    w2p, b2p = prep_fc(fc2_W, fc2_b, 4)
    w3p, b3p = prep_fc(fc3_W, fc3_b, 8)

    def prep_gin(w, b, s_in, s_out):
        wp = _pad2(w, s_in * 128, s_out * 128).reshape(s_in, 128,
                                                       s_out * 128)
        return wp, b.reshape(1, s_out * 128)

    g1w, g1b = prep_gin(gin1_W, gin1_b, 1, 4)
    g2w, g2b = prep_gin(gin2_W, gin2_b, 4, 8)
    g3w, g3b = prep_gin(gin3_W, gin3_b, 8, 4)

    def srcS(S):
        return (src[None, :] + (jnp.arange(S, dtype=jnp.int32) * N)[:, None]
                ).reshape(-1)

    h0 = _pad2(x1, N, 128)

    e1 = _proj(ea_p, w1p, b1p, 1).reshape(E, 128)
    agg1 = _make_conv(1)(h0, e1, src, dst).reshape(2, 1, N, 128)
    h1 = _dense(h0.reshape(1, N, 128), agg1, g1w, g1b,
                eps1.reshape(1, 1), 1, 4)

    e2 = _proj(ea_p, w2p, b2p, 4).reshape(4 * E, 128)
    agg2 = _make_conv(4)(h1.reshape(4 * N, 128), e2, srcS(4),
                         dst).reshape(2, 4, N, 128)
    h2 = _dense(h1, agg2, g2w, g2b, eps2.reshape(1, 1), 4, 8)

    e3 = _proj(ea_p, w3p, b3p, 8).reshape(8 * E, 128)
    agg3 = _make_conv(8)(h2.reshape(8 * N, 128), e3, srcS(8),
                         dst).reshape(2, 8, N, 128)
    h3 = _dense(h2, agg3, g3w, g3b, eps3.reshape(1, 1), 8, 4)

    bounds = jnp.searchsorted(batch, jnp.arange(G + 1)).astype(jnp.int32)
    bidx = jnp.minimum(4 * jnp.arange(34)[:, None] + jnp.arange(8)[None, :],
                       G)
    bounds = bounds[bidx].reshape(-1)

    xm = _make_segmax(4)(h3.reshape(4 * N, 128), bounds)
    return xm.reshape(G, 4, 8, 128)[:, :, 0, :].reshape(G, 512)


# speedup vs baseline: 1.4847x; 1.4847x over previous
import functools

import jax
import jax.numpy as jnp
from jax import lax
from jax.experimental import pallas as pl
from jax.experimental.pallas import tpu as pltpu
import jax.experimental.pallas.tpu_sc as plsc

N = 10000
E = 160000
G = 128
L = 16
NCORES = 2
NSUB = 16
CCH = 40
NT = 624
TAIL = N - NT * NSUB
ZR = 16


def _make_conv(S):
    EC = E // NCORES
    ET = EC // NSUB
    NCHUNK = ET // CCH
    mesh = plsc.VectorSubcoreMesh(core_axis_name="c", subcore_axis_name="s")

    @functools.partial(
        pl.kernel,
        out_type=jax.ShapeDtypeStruct((NCORES * S * N, 128), jnp.float32),
        mesh=mesh,
        scratch_types=[
            pltpu.VMEM((CCH,), jnp.int32),
            pltpu.VMEM((CCH,), jnp.int32),
            pltpu.VMEM((CCH,), jnp.int32),
            pltpu.VMEM((CCH,), jnp.int32),
            pltpu.VMEM((CCH, 128), jnp.float32),
            pltpu.VMEM((CCH, 128), jnp.float32),
            pltpu.VMEM((CCH, 128), jnp.float32),
            pltpu.VMEM((CCH, 128), jnp.float32),
            pltpu.VMEM((ZR, 128), jnp.float32),
            pltpu.VMEM_SHARED((N, 128), jnp.float32),
            pltpu.SemaphoreType.DMA,
            pltpu.SemaphoreType.DMA,
            pltpu.SemaphoreType.DMA,
            pltpu.SemaphoreType.DMA,
            pltpu.SemaphoreType.DMA,
            pltpu.SemaphoreType.DMA,
            pltpu.SemaphoreType.DMA,
            pltpu.SemaphoreType.DMA,
        ],
    )
    def conv(h_ref, e_ref, srcs_ref, dst_ref, out_ref,
             si0, si1, di0, di1, x0, x1, e0, e1, z_v, acc,
             is0, is1, gs0, gs1, es0, es1, ss0, ss1):
        si = [si0, si1]
        di = [di0, di1]
        x_v = [x0, x1]
        e_v = [e0, e1]
        isem = [is0, is1]
        gsem = [gs0, gs1]
        esem = [es0, es1]
        ssem = [ss0, ss1]
        c = lax.axis_index("c")
        t = lax.axis_index("s")
        zero16 = jnp.zeros((L,), jnp.float32)
        for r in range(ZR):
            for f in range(128 // L):
                z_v[r, pl.ds(f * L, L)] = zero16

        def slab_body(s, _):
            def zb(i, _):
                pltpu.sync_copy(z_v, acc.at[pl.ds(t * NT + i * ZR, ZR)])
                return 0
            lax.fori_loop(0, NT // ZR, zb, 0)

            @pl.when(t == NSUB - 1)
            def _():
                pltpu.sync_copy(z_v, acc.at[pl.ds(NT * NSUB, TAIL)])
            plsc.subcore_barrier()

            ebase0 = c * EC + t * ET

            def issue_idx(b, k):
                eb = ebase0 + k * CCH
                pltpu.async_copy(srcs_ref.at[pl.ds(s * E + eb, CCH)],
                                 si[b], isem[b])
                pltpu.async_copy(dst_ref.at[pl.ds(eb, CCH)], di[b], isem[b])

            def wait_idx(b):
                pltpu.make_async_copy(srcs_ref.at[pl.ds(0, CCH)], si[b],
                                      isem[b]).wait()
                pltpu.make_async_copy(dst_ref.at[pl.ds(0, CCH)], di[b],
                                      isem[b]).wait()

            def issue_gather(b, k):
                eb = ebase0 + k * CCH
                pltpu.async_copy(h_ref.at[si[b]], x_v[b], gsem[b])
                pltpu.async_copy(e_ref.at[pl.ds(s * E + eb, CCH)],
                                 e_v[b], esem[b])

            def wait_gather(b):
                pltpu.make_async_copy(h_ref.at[si[b]], x_v[b],
                                      gsem[b]).wait()
                pltpu.make_async_copy(e_ref.at[pl.ds(0, CCH)], e_v[b],
                                      esem[b]).wait()

            def do_chunk(b, k):
                nb = 1 - b

                @pl.when(k > 0)
                def _():
                    pltpu.make_async_copy(e_v[nb], acc.at[di[nb]],
                                          ssem[nb]).wait()

                @pl.when(k < NCHUNK - 1)
                def _():
                    issue_idx(nb, k + 1)
                wait_gather(b)
                for cc in range(CCH):
                    for f in range(128 // L):
                        sl = pl.ds(f * L, L)
                        e_v[b][cc, sl] = jnp.maximum(
                            x_v[b][cc, sl] + e_v[b][cc, sl], 0.0)

                @pl.when(k < NCHUNK - 1)
                def _():
                    wait_idx(nb)
                    issue_gather(nb, k + 1)
                pltpu.async_copy(e_v[b], acc.at[di[b]], ssem[b], add=True)

            issue_idx(0, 0)
            wait_idx(0)
            issue_gather(0, 0)

            def chunk_pair(p, _):
                k = p * 2
                do_chunk(0, k)
                do_chunk(1, k + 1)
                return 0
            lax.fori_loop(0, NCHUNK // 2, chunk_pair, 0)
            do_chunk(0, jnp.int32(NCHUNK - 1))
            pltpu.make_async_copy(e_v[0], acc.at[di[0]], ssem[0]).wait()
            plsc.subcore_barrier()

            ob0 = (c * S + s) * N
            pltpu.sync_copy(acc.at[pl.ds(t * NT, NT)],
                            out_ref.at[pl.ds(ob0 + t * NT, NT)])

            @pl.when(t == NSUB - 1)
            def _():
                pltpu.sync_copy(acc.at[pl.ds(NT * NSUB, TAIL)],
                                out_ref.at[pl.ds(ob0 + NT * NSUB, TAIL)])
            plsc.subcore_barrier()
            return 0
        lax.fori_loop(0, S, slab_body, 0)

    return conv


def _proj(ea_p, wp, bp, S):
    BE = 2000

    def body(ea_ref, w_ref, b_ref, o_ref):
        o_ref[0] = (jnp.dot(ea_ref[...], w_ref[0],
                            preferred_element_type=jnp.float32)
                    + b_ref[0])

    return pl.pallas_call(
        body,
        grid=(S, E // BE),
        in_specs=[
            pl.BlockSpec((BE, 16), lambda s, e: (e, 0)),
            pl.BlockSpec((1, 16, 128), lambda s, e: (s, 0, 0)),
            pl.BlockSpec((1, 1, 128), lambda s, e: (s, 0, 0)),
        ],
        out_specs=pl.BlockSpec((1, BE, 128), lambda s, e: (s, e, 0)),
        out_shape=jax.ShapeDtypeStruct((S, E, 128), jnp.float32),
    )(ea_p, wp, bp)


def _dense(h, agg, w, b, eps_arr, s_in, s_out):
    BN = 1000

    def body(eps_ref, h_ref, a_ref, w_ref, b_ref, o_ref):
        k = 1.0 + eps_ref[0, 0]
        acc = jnp.zeros((BN, s_out * 128), jnp.float32)
        for s in range(s_in):
            xs = h_ref[s] * k + a_ref[0, s] + a_ref[1, s]
            acc = acc + jnp.dot(xs, w_ref[s],
                                preferred_element_type=jnp.float32)
        z = acc + b_ref[0]
        z = jnp.where(z > 0, z, jnp.exp(jnp.minimum(z, 0.0)) - 1.0)
        for s2 in range(s_out):
            o_ref[s2] = z[:, s2 * 128:(s2 + 1) * 128]

    return pl.pallas_call(
        body,
        grid=(N // BN,),
        in_specs=[
            pl.BlockSpec(memory_space=pltpu.SMEM),
            pl.BlockSpec((s_in, BN, 128), lambda nb: (0, nb, 0)),
            pl.BlockSpec((2, s_in, BN, 128), lambda nb: (0, 0, nb, 0)),
            pl.BlockSpec((s_in, 128, s_out * 128), lambda nb: (0, 0, 0)),
            pl.BlockSpec((1, s_out * 128), lambda nb: (0, 0)),
        ],
        out_specs=pl.BlockSpec((s_out, BN, 128), lambda nb: (0, nb, 0)),
        out_shape=jax.ShapeDtypeStruct((s_out, N, 128), jnp.float32),
    )(eps_arr, h, agg, w, b)


def _make_segmax(S):
    gpw = G // (NCORES * NSUB)
    mesh = plsc.VectorSubcoreMesh(core_axis_name="c", subcore_axis_name="s")

    @functools.partial(
        pl.kernel,
        out_type=jax.ShapeDtypeStruct((G * S * 8, 128), jnp.float32),
        mesh=mesh,
        scratch_types=[
            pltpu.VMEM((272,), jnp.int32),
            pltpu.VMEM((8, 128), jnp.float32),
            pltpu.VMEM((8, 128), jnp.float32),
            pltpu.SemaphoreType.DMA,
        ],
    )
    def segmax(h_ref, b_ref, out_ref, bvm, row_v, acc_v, sem):
        c = lax.axis_index("c")
        t = lax.axis_index("s")
        wid = c * NSUB + t
        pltpu.sync_copy(b_ref, bvm)
        base = wid * gpw
        bvec = bvm[pl.ds(wid * 8, L)]

        def extract(j):
            return bvec[j]

        neg = jnp.full((L,), -jnp.inf, jnp.float32)
        for g in range(gpw):
            s0 = extract(g)
            s1 = extract(g + 1)
            s0a = (s0 // 8) * 8
            nch = (s1 - s0a + 7) // 8

            def slab_body(s, _):
                def chunk(i, accs):
                    r0 = jnp.minimum(s0a + i * 8, N - 8)
                    pltpu.async_copy(h_ref.at[pl.ds(s * N + r0, 8)],
                                     row_v, sem).wait()
                    out = list(accs)
                    for k in range(8):
                        rid = r0 + k
                        valid = jnp.logical_and(rid >= s0, rid < s1)
                        for f in range(128 // L):
                            v = jnp.where(valid, row_v[k, pl.ds(f * L, L)],
                                          neg)
                            out[f] = jnp.maximum(out[f], v)
                    return tuple(out)
                accs = lax.fori_loop(0, nch, chunk, (neg,) * (128 // L))
                for k in range(8):
                    for f in range(128 // L):
                        acc_v[k, pl.ds(f * L, L)] = accs[f]
                pltpu.sync_copy(
                    acc_v, out_ref.at[pl.ds(((base + g) * S + s) * 8, 8)])
                return 0
            lax.fori_loop(0, S, slab_body, 0)

    return segmax


def _pad2(a, rows, cols):
    return jnp.pad(a, ((0, rows - a.shape[0]), (0, cols - a.shape[1])))


def kernel(x1, edge_index, edge_attr, batch,
           fc1_W, fc1_b, fc2_W, fc2_b, fc3_W, fc3_b,
           gin1_W, gin1_b, eps1, gin2_W, gin2_b, eps2,
           gin3_W, gin3_b, eps3):
    src = edge_index[0].astype(jnp.int32)
    dst = edge_index[1].astype(jnp.int32)

    ea_p = _pad2(edge_attr, E, 16)

    def prep_fc(w, b, S):
        wp = _pad2(w, 16, S * 128).reshape(16, S, 128).transpose(1, 0, 2)
        bp = jnp.pad(b, (0, S * 128 - b.shape[0])).reshape(S, 1, 128)
        return wp, bp

    w1p, b1p = prep_fc(fc1_W, fc1_b, 1)
    w2p, b2p = prep_fc(fc2_W, fc2_b, 4)
    w3p, b3p = prep_fc(fc3_W, fc3_b, 8)

    def prep_gin(w, b, s_in, s_out):
        wp = _pad2(w, s_in * 128, s_out * 128).reshape(s_in, 128,
                                                       s_out * 128)
        return wp, b.reshape(1, s_out * 128)

    g1w, g1b = prep_gin(gin1_W, gin1_b, 1, 4)
    g2w, g2b = prep_gin(gin2_W, gin2_b, 4, 8)
    g3w, g3b = prep_gin(gin3_W, gin3_b, 8, 4)

    def srcS(S):
        return (src[None, :] + (jnp.arange(S, dtype=jnp.int32) * N)[:, None]
                ).reshape(-1)

    h0 = _pad2(x1, N, 128)

    e1 = _proj(ea_p, w1p, b1p, 1).reshape(E, 128)
    agg1 = _make_conv(1)(h0, e1, src, dst).reshape(2, 1, N, 128)
    h1 = _dense(h0.reshape(1, N, 128), agg1, g1w, g1b,
                eps1.reshape(1, 1), 1, 4)

    e2 = _proj(ea_p, w2p, b2p, 4).reshape(4 * E, 128)
    agg2 = _make_conv(4)(h1.reshape(4 * N, 128), e2, srcS(4),
                         dst).reshape(2, 4, N, 128)
    h2 = _dense(h1, agg2, g2w, g2b, eps2.reshape(1, 1), 4, 8)

    e3 = _proj(ea_p, w3p, b3p, 8).reshape(8 * E, 128)
    agg3 = _make_conv(8)(h2.reshape(8 * N, 128), e3, srcS(8),
                         dst).reshape(2, 8, N, 128)
    h3 = _dense(h2, agg3, g3w, g3b, eps3.reshape(1, 1), 8, 4)

    bounds = jnp.searchsorted(batch, jnp.arange(G + 1)).astype(jnp.int32)
    bidx = jnp.minimum(4 * jnp.arange(34)[:, None] + jnp.arange(8)[None, :],
                       G)
    bounds = bounds[bidx].reshape(-1)

    xm = _make_segmax(4)(h3.reshape(4 * N, 128), bounds)
    return xm.reshape(G, 4, 8, 128)[:, :, 0, :].reshape(G, 512)
